# baseline (speedup 1.0000x reference)
import functools
import math

import jax
import jax.numpy as jnp
from jax import lax
from jax.experimental import pallas as pl
from jax.experimental.pallas import tpu as pltpu
from jax.experimental.pallas import tpu_sc as plsc

EMB = 128
SCALE = math.sqrt(EMB)
LANES = 16
NBUF = 8


def _sc_embed(table, idx):
    nsent, sent = idx.shape
    info = plsc.get_sparse_core_info()
    nw = info.num_cores * info.num_subcores
    per_w = nsent // nw
    assert per_w % NBUF == 0

    mesh = plsc.VectorSubcoreMesh(core_axis_name="c", subcore_axis_name="s")

    @functools.partial(
        pl.kernel,
        mesh=mesh,
        out_type=jax.ShapeDtypeStruct((nsent, sent, EMB), jnp.float32),
        scratch_types=(
            [pltpu.VMEM((per_w, sent), jnp.int32)]
            + [pltpu.VMEM((sent, EMB), jnp.float32) for _ in range(NBUF)]
            + [pltpu.SemaphoreType.DMA, pltpu.SemaphoreType.DMA]
        ),
    )
    def k(table_hbm, idx_hbm, out_hbm, idx_v, *bufs_sems):
        bufs = list(bufs_sems[:NBUF])
        sem_in, sem_out = bufs_sems[NBUF:]
        wid = lax.axis_index("s") * info.num_cores + lax.axis_index("c")
        sbase = wid * per_w
        pltpu.sync_copy(idx_hbm.at[pl.ds(sbase, per_w)], idx_v)

        def gather(s, buf):
            return pltpu.make_async_copy(table_hbm.at[idx_v.at[s]], buf,
                                         sem_in)

        def out_copy(s, buf):
            return pltpu.make_async_copy(buf, out_hbm.at[sbase + s], sem_out)

        def scale(buf):
            @plsc.parallel_loop(0, sent, unroll=4)
            def row(r):
                for c in range(EMB // LANES):
                    sl = pl.ds(c * LANES, LANES)
                    buf[r, sl] = buf[r, sl] * SCALE

        for s in range(NBUF - 1):
            gather(s, bufs[s]).start()

        def outer(g, carry):
            for b in range(NBUF):
                s = g * NBUF + b
                gather(s, bufs[b]).wait()
                scale(bufs[b])

                @pl.when(s >= 1)
                def _():
                    out_copy(s - 1, bufs[(b - 1) % NBUF]).wait()

                out_copy(s, bufs[b]).start()

                @pl.when(s + NBUF - 1 < per_w)
                def _():
                    gather(s + NBUF - 1, bufs[(b + NBUF - 1) % NBUF]).start()

            return carry

        lax.fori_loop(0, per_w // NBUF, outer, 0)
        out_copy(per_w - 1, bufs[(per_w - 1) % NBUF]).wait()

    return k(table, idx)


def kernel(tokens, table):
    b, t = tokens.shape
    out = _sc_embed(table, tokens.astype(jnp.int32))
    return out


# speedup vs baseline: 1.7626x; 1.7626x over previous
import functools
import math

import jax
import jax.numpy as jnp
from jax import lax
from jax.experimental import pallas as pl
from jax.experimental.pallas import tpu as pltpu
from jax.experimental.pallas import tpu_sc as plsc

EMB = 128
SCALE = math.sqrt(EMB)
LANES = 16
CHUNK = 128
NBUF = 5


def _sc_embed(total, table, idx3d):
    info = plsc.get_sparse_core_info()
    nw = info.num_cores * info.num_subcores
    per_w = total // nw
    chunks = per_w // CHUNK
    assert chunks % NBUF == 0

    mesh = plsc.VectorSubcoreMesh(core_axis_name="c", subcore_axis_name="s")

    @functools.partial(
        pl.kernel,
        mesh=mesh,
        out_type=jax.ShapeDtypeStruct((total, EMB), jnp.float32),
        scratch_types=(
            [pltpu.VMEM((chunks, CHUNK), jnp.int32)]
            + [pltpu.VMEM((CHUNK, EMB), jnp.float32) for _ in range(NBUF)]
            + [pltpu.SemaphoreType.DMA, pltpu.SemaphoreType.DMA]
        ),
    )
    def k(table_hbm, idx_hbm, out_hbm, idx_v, *bufs_sems):
        bufs = list(bufs_sems[:NBUF])
        sem_in, sem_out = bufs_sems[NBUF:]
        wid = lax.axis_index("s") * info.num_cores + lax.axis_index("c")
        base = wid * per_w
        pltpu.sync_copy(idx_hbm.at[wid], idx_v)

        def gather(j, buf):
            return pltpu.make_async_copy(table_hbm.at[idx_v.at[j]], buf,
                                         sem_in)

        def out_copy(j, buf):
            return pltpu.make_async_copy(
                buf, out_hbm.at[pl.ds(base + j * CHUNK, CHUNK)], sem_out)

        def scale(buf):
            @plsc.parallel_loop(0, CHUNK, unroll=4)
            def row(r):
                for c in range(EMB // LANES):
                    sl = pl.ds(c * LANES, LANES)
                    buf[r, sl] = buf[r, sl] * SCALE

        for j in range(NBUF - 1):
            gather(j, bufs[j]).start()

        def outer(g, carry):
            for b in range(NBUF):
                j = g * NBUF + b
                gather(j, bufs[b]).wait()
                scale(bufs[b])

                @pl.when(j >= 1)
                def _():
                    out_copy(j - 1, bufs[(b - 1) % NBUF]).wait()

                out_copy(j, bufs[b]).start()

                @pl.when(j + NBUF - 1 < chunks)
                def _():
                    gather(j + NBUF - 1, bufs[(b + NBUF - 1) % NBUF]).start()

            return carry

        lax.fori_loop(0, chunks // NBUF, outer, 0)
        out_copy(chunks - 1, bufs[(chunks - 1) % NBUF]).wait()

    return k(table, idx3d)


def kernel(tokens, table):
    b, t = tokens.shape
    total = b * t
    info = plsc.get_sparse_core_info()
    nw = info.num_cores * info.num_subcores
    idx3d = tokens.astype(jnp.int32).T.reshape(nw, total // (nw * CHUNK), CHUNK)
    flat = _sc_embed(total, table, idx3d)
    return flat.reshape(t, b, EMB).transpose(1, 0, 2)
